# baseline (speedup 1.0000x reference)
import functools

import jax
import jax.numpy as jnp
from jax import lax
from jax.experimental import pallas as pl
from jax.experimental.pallas import tpu as pltpu
from jax.experimental.pallas import tpu_sc as plsc

N = 8192
D = 4096
E = 8
H = 512
BM = 256
NT = N // 2
MB_H = NT // BM
NG = 64
GB = MB_H + 28
GBP = 64
T_MAX = GB * BM

NW = 32
CH = 8
TPW1 = NT // NW
NCH1 = TPW1 // CH
TPW3 = N // NW
NCH3 = TPW3 // CH
MB_ALL = N // BM


def _tc1a_kernel(x_ref, wg_ref, bg_ref,
                 g_ref, rank_ref, ps_ref, hist_ref, psum_ref,
                 carry_ref, pacc_ref):
    m = pl.program_id(0)
    x_blk = x_ref[...]

    logits = lax.dot_general(
        x_blk.astype(jnp.bfloat16), wg_ref[...].astype(jnp.bfloat16),
        dimension_numbers=(((1,), (0,)), ((), ())),
        preferred_element_type=jnp.float32) + bg_ref[...]
    mx = jnp.max(logits, axis=1, keepdims=True)
    ex = jnp.exp(logits - mx)
    p = ex / jnp.sum(ex, axis=1, keepdims=True)

    idx = lax.broadcasted_iota(jnp.int32, (BM, E), 1)
    v1 = jnp.max(p, axis=1, keepdims=True)
    i1 = jnp.min(jnp.where(p == v1, idx, E), axis=1, keepdims=True)
    p_m = jnp.where(idx == i1, -1.0, p)
    v2 = jnp.max(p_m, axis=1, keepdims=True)
    i2 = jnp.min(jnp.where(p_m == v2, idx, E), axis=1, keepdims=True)

    lo = jnp.minimum(i1, i2)
    hi = jnp.maximum(i1, i2)
    p_lo = jnp.where(i1 < i2, v1, v2)
    p_hi = jnp.where(i1 < i2, v2, v1)
    g = lo * E + hi

    @pl.when(m == 0)
    def _init():
        carry_ref[...] = jnp.zeros_like(carry_ref)
        pacc_ref[...] = jnp.zeros_like(pacc_ref)

    pacc_ref[...] += jnp.sum(p, axis=0, keepdims=True)

    gcols = lax.broadcasted_iota(jnp.int32, (BM, NG), 1)
    ind = (g == gcols).astype(jnp.float32)
    tri = (lax.broadcasted_iota(jnp.int32, (BM, BM), 0) >
           lax.broadcasted_iota(jnp.int32, (BM, BM), 1)).astype(jnp.bfloat16)
    cnt_before = lax.dot_general(
        tri, ind.astype(jnp.bfloat16),
        dimension_numbers=(((1,), (0,)), ((), ())),
        preferred_element_type=jnp.float32)
    rank = jnp.sum((carry_ref[...] + cnt_before) * ind, axis=1, keepdims=True)
    carry_ref[...] += jnp.sum(ind, axis=0, keepdims=True)

    g_ref[...] = g
    rank_ref[...] = rank.astype(jnp.int32)
    ps_ref[...] = jnp.concatenate(
        [p_lo, p_hi, jnp.zeros((BM, 126), jnp.float32)], axis=1)

    @pl.when(m == MB_H - 1)
    def _fin():
        hist_ref[...] = carry_ref[...]
        psum_ref[...] = pacc_ref[...]


def _tc1a(xh, Wg, bg2):
    return pl.pallas_call(
        _tc1a_kernel,
        grid=(MB_H,),
        in_specs=[
            pl.BlockSpec((BM, D), lambda m: (m, 0)),
            pl.BlockSpec((D, E), lambda m: (0, 0)),
            pl.BlockSpec((1, E), lambda m: (0, 0)),
        ],
        out_specs=[
            pl.BlockSpec((BM, 1), lambda m: (m, 0)),
            pl.BlockSpec((BM, 1), lambda m: (m, 0)),
            pl.BlockSpec((BM, 128), lambda m: (m, 0)),
            pl.BlockSpec((1, NG), lambda m: (0, 0)),
            pl.BlockSpec((1, E), lambda m: (0, 0)),
        ],
        out_shape=[
            jax.ShapeDtypeStruct((NT, 1), jnp.int32),
            jax.ShapeDtypeStruct((NT, 1), jnp.int32),
            jax.ShapeDtypeStruct((NT, 128), jnp.float32),
            jax.ShapeDtypeStruct((1, NG), jnp.float32),
            jax.ShapeDtypeStruct((1, E), jnp.float32),
        ],
        scratch_shapes=[
            pltpu.VMEM((1, NG), jnp.float32),
            pltpu.VMEM((1, E), jnp.float32),
        ],
        compiler_params=pltpu.CompilerParams(
            dimension_semantics=("arbitrary",)),
    )(xh, Wg, bg2)


def _tc1b_kernel(g_ref, rank_ref, hist_ref, q_ref, ea_ref, eb_ref, nact_ref):
    m = pl.program_id(0)
    hist = hist_ref[...]
    nb = jnp.floor((hist + (BM - 1)) / BM)
    ut = (lax.broadcasted_iota(jnp.int32, (NG, NG), 0) <
          lax.broadcasted_iota(jnp.int32, (NG, NG), 1)).astype(jnp.bfloat16)
    bstart = lax.dot_general(
        nb.astype(jnp.bfloat16), ut,
        dimension_numbers=(((1,), (0,)), ((), ())),
        preferred_element_type=jnp.float32)

    g = g_ref[...]
    gcols = lax.broadcasted_iota(jnp.int32, (BM, NG), 1)
    ind = (g == gcols).astype(jnp.float32)
    qpos = (jnp.sum(ind * bstart, axis=1, keepdims=True) * BM
            + rank_ref[...].astype(jnp.float32))
    q_ref[...] = qpos.astype(jnp.int32)

    @pl.when(m == 0)
    def _desc():
        brow = lax.broadcasted_iota(jnp.int32, (GBP, NG), 0).astype(jnp.float32)
        act = jnp.logical_and(brow >= bstart, brow < bstart + nb)
        actf = act.astype(jnp.float32)
        glane = lax.broadcasted_iota(jnp.int32, (GBP, NG), 1).astype(jnp.float32)
        gb = jnp.sum(actf * glane, axis=1, keepdims=True)
        ea = jnp.floor(gb / E)
        eb = gb - ea * E
        na = jnp.sum(actf * jnp.clip(hist - (brow - bstart) * BM, 0.0, 1.0 * BM),
                     axis=1, keepdims=True)
        ea_ref[...] = ea.astype(jnp.int32)
        eb_ref[...] = eb.astype(jnp.int32)
        nact_ref[...] = na.astype(jnp.int32)


def _tc1b(g, rank, hist):
    return pl.pallas_call(
        _tc1b_kernel,
        grid=(MB_H,),
        in_specs=[
            pl.BlockSpec((BM, 1), lambda m: (m, 0)),
            pl.BlockSpec((BM, 1), lambda m: (m, 0)),
            pl.BlockSpec((1, NG), lambda m: (0, 0)),
        ],
        out_specs=[
            pl.BlockSpec((BM, 1), lambda m: (m, 0)),
            pl.BlockSpec((GBP, 1), lambda m: (0, 0)),
            pl.BlockSpec((GBP, 1), lambda m: (0, 0)),
            pl.BlockSpec((GBP, 1), lambda m: (0, 0)),
        ],
        out_shape=[
            jax.ShapeDtypeStruct((NT, 1), jnp.int32),
            jax.ShapeDtypeStruct((GBP, 1), jnp.int32),
            jax.ShapeDtypeStruct((GBP, 1), jnp.int32),
            jax.ShapeDtypeStruct((GBP, 1), jnp.int32),
        ],
        compiler_params=pltpu.CompilerParams(
            dimension_semantics=("arbitrary",)),
    )(g, rank, hist)


@functools.cache
def _make_sc_scatter():
    @functools.partial(
        pl.kernel,
        out_type=[
            jax.ShapeDtypeStruct((T_MAX, D), jnp.float32),
            jax.ShapeDtypeStruct((T_MAX, 128), jnp.float32),
        ],
        mesh=plsc.VectorSubcoreMesh(core_axis_name="c", subcore_axis_name="s",
                                    num_cores=2, num_subcores=16),
        scratch_types=[
            pltpu.VMEM((NCH1, CH), jnp.int32),
            pltpu.VMEM((2, CH, D), jnp.float32),
            pltpu.VMEM((2, CH, 128), jnp.float32),
            pltpu.SemaphoreType.DMA,
            pltpu.SemaphoreType.DMA,
            pltpu.SemaphoreType.DMA,
            pltpu.SemaphoreType.DMA,
            pltpu.SemaphoreType.DMA,
        ],
    )
    def _sc_scatter(x_hbm, ps_hbm, q_hbm, xs_out, pss_out, qv, bufx, bufp,
                    semq, semld, semlp, semx, semp):
        wid = lax.axis_index("s") * 2 + lax.axis_index("c")
        base = wid * TPW1
        cq = pltpu.async_copy(q_hbm.at[pl.ds(wid * NCH1, NCH1)], qv, semq)

        def load(i):
            b = i % 2
            off = base + i * CH
            pltpu.async_copy(x_hbm.at[pl.ds(off, CH)], bufx.at[b], semld)
            pltpu.async_copy(ps_hbm.at[pl.ds(off, CH)], bufp.at[b], semlp)

        load(0)
        cq.wait()
        for i in range(NCH1):
            b = i % 2
            pltpu.make_async_copy(x_hbm.at[pl.ds(0, CH)], bufx.at[b],
                                  semld).wait()
            pltpu.make_async_copy(ps_hbm.at[pl.ds(0, CH)], bufp.at[b],
                                  semlp).wait()
            cx = pltpu.async_copy(bufx.at[b], xs_out.at[qv.at[i]], semx)
            cp = pltpu.async_copy(bufp.at[b], pss_out.at[qv.at[i]], semp)
            if i + 1 < NCH1:
                load(i + 1)
            cx.wait()
            cp.wait()

    return _sc_scatter


def _tc2_kernel(xs_ref, ps_ref, ea_ref, eb_ref, nact_ref, we_hbm, be_ref,
                comb_ref, we_vmem, wtmp, sem):
    gi = pl.program_id(0)

    @pl.when(gi == 0)
    def _load():
        for e in range(E):
            cp = pltpu.make_async_copy(we_hbm.at[e], wtmp, sem)
            cp.start()
            cp.wait()
            we_vmem[e] = wtmp[...].astype(jnp.bfloat16)

    nact = nact_ref[gi, 0]

    @pl.when(nact > 0)
    def _compute():
        ea = ea_ref[gi, 0]
        eb = eb_ref[gi, 0]
        xsb = xs_ref[...].astype(jnp.bfloat16)
        ps = ps_ref[...]
        pa = ps[:, 0:1]
        pb = ps[:, 1:2]
        mma = lax.dot_general(
            xsb, we_vmem[ea],
            dimension_numbers=(((1,), (0,)), ((), ())),
            preferred_element_type=jnp.float32)
        mmb = lax.dot_general(
            xsb, we_vmem[eb],
            dimension_numbers=(((1,), (0,)), ((), ())),
            preferred_element_type=jnp.float32)
        comb_ref[...] = (pa * (mma + be_ref[ea][None, :])
                         + pb * (mmb + be_ref[eb][None, :]))


def _tc2(xs, pss, ea, eb, nact, We, be):
    return pl.pallas_call(
        _tc2_kernel,
        grid=(GB,),
        in_specs=[
            pl.BlockSpec((BM, D), lambda m: (m, 0)),
            pl.BlockSpec((BM, 128), lambda m: (m, 0)),
            pl.BlockSpec(memory_space=pltpu.MemorySpace.SMEM),
            pl.BlockSpec(memory_space=pltpu.MemorySpace.SMEM),
            pl.BlockSpec(memory_space=pltpu.MemorySpace.SMEM),
            pl.BlockSpec(memory_space=pl.ANY),
            pl.BlockSpec((E, H), lambda m: (0, 0)),
        ],
        out_specs=pl.BlockSpec((BM, H), lambda m: (m, 0)),
        out_shape=jax.ShapeDtypeStruct((T_MAX, H), jnp.float32),
        scratch_shapes=[
            pltpu.VMEM((E, D, H), jnp.bfloat16),
            pltpu.VMEM((D, H), jnp.float32),
            pltpu.SemaphoreType.DMA,
        ],
        compiler_params=pltpu.CompilerParams(
            dimension_semantics=("arbitrary",)),
    )(xs, pss, ea, eb, nact, We, be)


@functools.cache
def _make_sc_gather():
    @functools.partial(
        pl.kernel,
        out_type=jax.ShapeDtypeStruct((N, H), jnp.float32),
        mesh=plsc.VectorSubcoreMesh(core_axis_name="c", subcore_axis_name="s",
                                    num_cores=2, num_subcores=16),
        scratch_types=[
            pltpu.VMEM((NCH3, CH), jnp.int32),
            pltpu.VMEM((2, CH, H), jnp.float32),
            pltpu.SemaphoreType.DMA,
            pltpu.SemaphoreType.DMA,
            pltpu.SemaphoreType.DMA,
            pltpu.SemaphoreType.DMA,
        ],
    )
    def _sc_gather(comb_a, comb_b, q_hbm, outs, qv, bufc, semq, semg,
                   semw0, semw1):
        wid = lax.axis_index("s") * 2 + lax.axis_index("c")
        base = wid * TPW3
        semw = (semw0, semw1)
        cq = pltpu.async_copy(q_hbm.at[pl.ds(wid * NCH3, NCH3)], qv, semq)
        cq.wait()

        def ring(comb_hbm):
            pltpu.async_copy(comb_hbm.at[qv.at[0]], bufc.at[0], semg)
            for i in range(NCH3):
                b = i % 2
                pltpu.make_async_copy(comb_hbm.at[pl.ds(0, CH)], bufc.at[b],
                                      semg).wait()
                pltpu.async_copy(bufc.at[b],
                                 outs.at[pl.ds(base + i * CH, CH)], semw[b])
                if i + 1 < NCH3:
                    if i >= 1:
                        pltpu.make_async_copy(
                            comb_hbm.at[pl.ds(0, CH)], bufc.at[1 - b],
                            semw[1 - b]).wait()
                    pltpu.async_copy(comb_hbm.at[qv.at[i + 1]],
                                     bufc.at[1 - b], semg)
            for b in (NCH3 % 2, (NCH3 + 1) % 2):
                pltpu.make_async_copy(comb_hbm.at[pl.ds(0, CH)], bufc.at[b],
                                      semw[b]).wait()

        @pl.when(wid < NW // 2)
        def _half_a():
            ring(comb_a)

        @pl.when(wid >= NW // 2)
        def _half_b():
            ring(comb_b)

    return _sc_gather


def _tc3_kernel(os_ref, out_ref):
    out_ref[...] = jnp.tile(os_ref[...], (1, E))


def _tc3(outs):
    return pl.pallas_call(
        _tc3_kernel,
        grid=(MB_ALL,),
        in_specs=[pl.BlockSpec((BM, H), lambda m: (m, 0))],
        out_specs=pl.BlockSpec((BM, E * H), lambda m: (m, 0)),
        out_shape=jax.ShapeDtypeStruct((N, E * H), jnp.float32),
    )(outs)


@jax.jit
def kernel(x, Wg, bg, We, be):
    bg2 = bg.reshape(1, E)
    combs, qs, psums = [], [], []
    for h in range(2):
        xh = lax.slice_in_dim(x, h * NT, (h + 1) * NT, axis=0)
        g, rank, ps_tok, hist, psum = _tc1a(xh, Wg, bg2)
        q2, ea, eb, nact = _tc1b(g, rank, hist)
        q = q2.reshape(NW * NCH1, CH)
        xs, pss = _make_sc_scatter()(xh, ps_tok, q)
        comb = _tc2(xs, pss, ea, eb, nact, We, be)
        combs.append(comb)
        qs.append(q2)
        psums.append(psum)
    q_all = jnp.concatenate(qs, axis=0).reshape(N // CH, CH)
    outs = _make_sc_gather()(combs[0], combs[1], q_all)
    out = _tc3(outs)
    mean_p = (psums[0] + psums[1]).reshape(E) / N
    loss = jnp.sum((mean_p - 1.0 / E) ** 2)
    return out, loss


# speedup vs baseline: 1.2922x; 1.2922x over previous
import functools

import jax
import jax.numpy as jnp
from jax import lax
from jax.experimental import pallas as pl
from jax.experimental.pallas import tpu as pltpu
from jax.experimental.pallas import tpu_sc as plsc

N = 8192
D = 4096
E = 8
H = 512
BM = 256
M_BLOCKS = N // BM
NG = 64
GB = 60
GBP = 64
T_MAX = GB * BM

NW = 32
TPW = N // NW
CH = 8
NCH = TPW // CH


def _tc1a_kernel(x_ref, wg_ref, bg_ref,
                 g_ref, rank_ref, ps_ref, xb_ref, hist_ref, loss_ref,
                 carry_ref, psum_ref):
    m = pl.program_id(0)
    x_blk = x_ref[...]
    xb_blk = x_blk.astype(jnp.bfloat16)
    xb_ref[...] = pltpu.bitcast(
        xb_blk.reshape(2 * BM, D // 2), jnp.int32)

    logits = lax.dot_general(
        xb_blk, wg_ref[...].astype(jnp.bfloat16),
        dimension_numbers=(((1,), (0,)), ((), ())),
        preferred_element_type=jnp.float32) + bg_ref[...]
    mx = jnp.max(logits, axis=1, keepdims=True)
    ex = jnp.exp(logits - mx)
    p = ex / jnp.sum(ex, axis=1, keepdims=True)

    idx = lax.broadcasted_iota(jnp.int32, (BM, E), 1)
    v1 = jnp.max(p, axis=1, keepdims=True)
    i1 = jnp.min(jnp.where(p == v1, idx, E), axis=1, keepdims=True)
    p_m = jnp.where(idx == i1, -1.0, p)
    v2 = jnp.max(p_m, axis=1, keepdims=True)
    i2 = jnp.min(jnp.where(p_m == v2, idx, E), axis=1, keepdims=True)

    lo = jnp.minimum(i1, i2)
    hi = jnp.maximum(i1, i2)
    p_lo = jnp.where(i1 < i2, v1, v2)
    p_hi = jnp.where(i1 < i2, v2, v1)
    g = lo * E + hi

    @pl.when(m == 0)
    def _init():
        carry_ref[...] = jnp.zeros_like(carry_ref)
        psum_ref[...] = jnp.zeros_like(psum_ref)

    psum_ref[...] += jnp.sum(p, axis=0, keepdims=True)

    gcols = lax.broadcasted_iota(jnp.int32, (BM, NG), 1)
    ind = (g == gcols).astype(jnp.float32)
    tri = (lax.broadcasted_iota(jnp.int32, (BM, BM), 0) >
           lax.broadcasted_iota(jnp.int32, (BM, BM), 1)).astype(jnp.bfloat16)
    cnt_before = lax.dot_general(
        tri, ind.astype(jnp.bfloat16),
        dimension_numbers=(((1,), (0,)), ((), ())),
        preferred_element_type=jnp.float32)
    rank = jnp.sum((carry_ref[...] + cnt_before) * ind, axis=1, keepdims=True)
    carry_ref[...] += jnp.sum(ind, axis=0, keepdims=True)

    g_ref[...] = g
    rank_ref[...] = rank.astype(jnp.int32)
    ps_ref[...] = jnp.concatenate(
        [p_lo, p_hi, jnp.zeros((BM, 126), jnp.float32)], axis=1)

    @pl.when(m == M_BLOCKS - 1)
    def _fin():
        hist_ref[...] = carry_ref[...]
        mean_p = psum_ref[...] / N
        loss_ref[...] = jnp.sum((mean_p - 1.0 / E) ** 2, keepdims=True)


def _tc1a(x, Wg, bg2):
    return pl.pallas_call(
        _tc1a_kernel,
        grid=(M_BLOCKS,),
        in_specs=[
            pl.BlockSpec((BM, D), lambda m: (m, 0)),
            pl.BlockSpec((D, E), lambda m: (0, 0)),
            pl.BlockSpec((1, E), lambda m: (0, 0)),
        ],
        out_specs=[
            pl.BlockSpec((BM, 1), lambda m: (m, 0)),
            pl.BlockSpec((BM, 1), lambda m: (m, 0)),
            pl.BlockSpec((BM, 128), lambda m: (m, 0)),
            pl.BlockSpec((BM, D // 2), lambda m: (m, 0)),
            pl.BlockSpec((1, NG), lambda m: (0, 0)),
            pl.BlockSpec((1, 1), lambda m: (0, 0)),
        ],
        out_shape=[
            jax.ShapeDtypeStruct((N, 1), jnp.int32),
            jax.ShapeDtypeStruct((N, 1), jnp.int32),
            jax.ShapeDtypeStruct((N, 128), jnp.float32),
            jax.ShapeDtypeStruct((N, D // 2), jnp.int32),
            jax.ShapeDtypeStruct((1, NG), jnp.float32),
            jax.ShapeDtypeStruct((1, 1), jnp.float32),
        ],
        scratch_shapes=[
            pltpu.VMEM((1, NG), jnp.float32),
            pltpu.VMEM((1, E), jnp.float32),
        ],
        compiler_params=pltpu.CompilerParams(
            dimension_semantics=("arbitrary",)),
    )(x, Wg, bg2)


def _tc1b_kernel(g_ref, rank_ref, hist_ref, q_ref, ea_ref, eb_ref, nact_ref):
    m = pl.program_id(0)
    hist = hist_ref[...]
    nb = jnp.floor((hist + (BM - 1)) / BM)
    ut = (lax.broadcasted_iota(jnp.int32, (NG, NG), 0) <
          lax.broadcasted_iota(jnp.int32, (NG, NG), 1)).astype(jnp.bfloat16)
    bstart = lax.dot_general(
        nb.astype(jnp.bfloat16), ut,
        dimension_numbers=(((1,), (0,)), ((), ())),
        preferred_element_type=jnp.float32)

    g = g_ref[...]
    gcols = lax.broadcasted_iota(jnp.int32, (BM, NG), 1)
    ind = (g == gcols).astype(jnp.float32)
    qpos = (jnp.sum(ind * bstart, axis=1, keepdims=True) * BM
            + rank_ref[...].astype(jnp.float32))
    q_ref[...] = qpos.astype(jnp.int32)

    @pl.when(m == 0)
    def _desc():
        brow = lax.broadcasted_iota(jnp.int32, (GBP, NG), 0).astype(jnp.float32)
        act = jnp.logical_and(brow >= bstart, brow < bstart + nb)
        actf = act.astype(jnp.float32)
        glane = lax.broadcasted_iota(jnp.int32, (GBP, NG), 1).astype(jnp.float32)
        gb = jnp.sum(actf * glane, axis=1, keepdims=True)
        ea = jnp.floor(gb / E)
        eb = gb - ea * E
        na = jnp.sum(actf * jnp.clip(hist - (brow - bstart) * BM, 0.0, 1.0 * BM),
                     axis=1, keepdims=True)
        ea_ref[...] = ea.astype(jnp.int32)
        eb_ref[...] = eb.astype(jnp.int32)
        nact_ref[...] = na.astype(jnp.int32)


def _tc1b(g, rank, hist):
    return pl.pallas_call(
        _tc1b_kernel,
        grid=(M_BLOCKS,),
        in_specs=[
            pl.BlockSpec((BM, 1), lambda m: (m, 0)),
            pl.BlockSpec((BM, 1), lambda m: (m, 0)),
            pl.BlockSpec((1, NG), lambda m: (0, 0)),
        ],
        out_specs=[
            pl.BlockSpec((BM, 1), lambda m: (m, 0)),
            pl.BlockSpec((GBP, 1), lambda m: (0, 0)),
            pl.BlockSpec((GBP, 1), lambda m: (0, 0)),
            pl.BlockSpec((GBP, 1), lambda m: (0, 0)),
        ],
        out_shape=[
            jax.ShapeDtypeStruct((N, 1), jnp.int32),
            jax.ShapeDtypeStruct((GBP, 1), jnp.int32),
            jax.ShapeDtypeStruct((GBP, 1), jnp.int32),
            jax.ShapeDtypeStruct((GBP, 1), jnp.int32),
        ],
        compiler_params=pltpu.CompilerParams(
            dimension_semantics=("arbitrary",)),
    )(g, rank, hist)


@functools.cache
def _make_sc_scatter():
    @functools.partial(
        pl.kernel,
        out_type=[
            jax.ShapeDtypeStruct((T_MAX, D // 2), jnp.int32),
            jax.ShapeDtypeStruct((T_MAX, 128), jnp.float32),
        ],
        mesh=plsc.VectorSubcoreMesh(core_axis_name="c", subcore_axis_name="s",
                                    num_cores=2, num_subcores=16),
        scratch_types=[
            pltpu.VMEM((NCH, CH), jnp.int32),
            pltpu.VMEM((2, CH, D // 2), jnp.int32),
            pltpu.VMEM((2, CH, 128), jnp.float32),
            pltpu.SemaphoreType.DMA,
            pltpu.SemaphoreType.DMA,
            pltpu.SemaphoreType.DMA,
            pltpu.SemaphoreType.DMA,
            pltpu.SemaphoreType.DMA,
        ],
    )
    def _sc_scatter(x_hbm, ps_hbm, q_hbm, xs_out, pss_out, qv, bufx, bufp,
                    semq, semld, semlp, semx, semp):
        wid = lax.axis_index("s") * 2 + lax.axis_index("c")
        base = wid * TPW
        cq = pltpu.async_copy(q_hbm.at[pl.ds(wid * NCH, NCH)], qv, semq)

        def load(i):
            b = i % 2
            off = base + i * CH
            pltpu.async_copy(x_hbm.at[pl.ds(off, CH)], bufx.at[b], semld)
            pltpu.async_copy(ps_hbm.at[pl.ds(off, CH)], bufp.at[b], semlp)

        load(0)
        cq.wait()
        for i in range(NCH):
            b = i % 2
            pltpu.make_async_copy(x_hbm.at[pl.ds(0, CH)], bufx.at[b],
                                  semld).wait()
            pltpu.make_async_copy(ps_hbm.at[pl.ds(0, CH)], bufp.at[b],
                                  semlp).wait()
            cx = pltpu.async_copy(bufx.at[b], xs_out.at[qv.at[i]], semx)
            cp = pltpu.async_copy(bufp.at[b], pss_out.at[qv.at[i]], semp)
            if i + 1 < NCH:
                load(i + 1)
            cx.wait()
            cp.wait()

    return _sc_scatter


def _tc2_kernel(xs_ref, ps_ref, ea_ref, eb_ref, nact_ref, we_hbm, be_ref,
                comb_ref, we_vmem, wtmp, sem):
    gi = pl.program_id(0)

    @pl.when(gi == 0)
    def _load():
        for e in range(E):
            cp = pltpu.make_async_copy(we_hbm.at[e], wtmp, sem)
            cp.start()
            cp.wait()
            we_vmem[e] = wtmp[...].astype(jnp.bfloat16)

    nact = nact_ref[gi, 0]

    @pl.when(nact > 0)
    def _compute():
        ea = ea_ref[gi, 0]
        eb = eb_ref[gi, 0]
        xsb = pltpu.bitcast(
            xs_ref[...], jnp.bfloat16).reshape(BM, D)
        ps = ps_ref[...]
        pa = ps[:, 0:1]
        pb = ps[:, 1:2]

        mma = lax.dot_general(
            xsb, we_vmem[ea],
            dimension_numbers=(((1,), (0,)), ((), ())),
            preferred_element_type=jnp.float32)
        mmb = lax.dot_general(
            xsb, we_vmem[eb],
            dimension_numbers=(((1,), (0,)), ((), ())),
            preferred_element_type=jnp.float32)
        comb_ref[...] = (pa * (mma + be_ref[ea][None, :])
                         + pb * (mmb + be_ref[eb][None, :]))


def _tc2(xs, pss, ea, eb, nact, We, be):
    return pl.pallas_call(
        _tc2_kernel,
        grid=(GB,),
        in_specs=[
            pl.BlockSpec((BM, D // 2), lambda m: (m, 0)),
            pl.BlockSpec((BM, 128), lambda m: (m, 0)),
            pl.BlockSpec(memory_space=pltpu.MemorySpace.SMEM),
            pl.BlockSpec(memory_space=pltpu.MemorySpace.SMEM),
            pl.BlockSpec(memory_space=pltpu.MemorySpace.SMEM),
            pl.BlockSpec(memory_space=pl.ANY),
            pl.BlockSpec((E, H), lambda m: (0, 0)),
        ],
        out_specs=pl.BlockSpec((BM, H), lambda m: (m, 0)),
        out_shape=jax.ShapeDtypeStruct((T_MAX, H), jnp.float32),
        scratch_shapes=[
            pltpu.VMEM((E, D, H), jnp.bfloat16),
            pltpu.VMEM((D, H), jnp.float32),
            pltpu.SemaphoreType.DMA,
        ],
        compiler_params=pltpu.CompilerParams(
            dimension_semantics=("arbitrary",)),
    )(xs, pss, ea, eb, nact, We, be)


@functools.cache
def _make_sc_gather():
    @functools.partial(
        pl.kernel,
        out_type=jax.ShapeDtypeStruct((N, H), jnp.float32),
        mesh=plsc.VectorSubcoreMesh(core_axis_name="c", subcore_axis_name="s",
                                    num_cores=2, num_subcores=16),
        scratch_types=[
            pltpu.VMEM((NCH, CH), jnp.int32),
            pltpu.VMEM((2, CH, H), jnp.float32),
            pltpu.SemaphoreType.DMA,
            pltpu.SemaphoreType.DMA,
            pltpu.SemaphoreType.DMA,
            pltpu.SemaphoreType.DMA,
        ],
    )
    def _sc_gather(comb_hbm, q_hbm, outs, qv, bufc, semq, semg, semw0, semw1):
        wid = lax.axis_index("s") * 2 + lax.axis_index("c")
        base = wid * TPW
        semw = (semw0, semw1)
        cq = pltpu.async_copy(q_hbm.at[pl.ds(wid * NCH, NCH)], qv, semq)
        cq.wait()
        pltpu.async_copy(comb_hbm.at[qv.at[0]], bufc.at[0], semg)
        for i in range(NCH):
            b = i % 2
            pltpu.make_async_copy(comb_hbm.at[pl.ds(0, CH)], bufc.at[b],
                                  semg).wait()
            pltpu.async_copy(bufc.at[b], outs.at[pl.ds(base + i * CH, CH)],
                             semw[b])
            if i + 1 < NCH:
                if i >= 1:
                    pltpu.make_async_copy(
                        comb_hbm.at[pl.ds(0, CH)], bufc.at[1 - b],
                        semw[1 - b]).wait()
                pltpu.async_copy(comb_hbm.at[qv.at[i + 1]], bufc.at[1 - b],
                                 semg)
        for b in (NCH % 2, (NCH + 1) % 2):
            pltpu.make_async_copy(comb_hbm.at[pl.ds(0, CH)], bufc.at[b],
                                  semw[b]).wait()

    return _sc_gather


def _tc3_kernel(os_ref, out_ref):
    out_ref[...] = jnp.tile(os_ref[...], (1, E))


def _tc3(outs):
    return pl.pallas_call(
        _tc3_kernel,
        grid=(M_BLOCKS,),
        in_specs=[pl.BlockSpec((BM, H), lambda m: (m, 0))],
        out_specs=pl.BlockSpec((BM, E * H), lambda m: (m, 0)),
        out_shape=jax.ShapeDtypeStruct((N, E * H), jnp.float32),
    )(outs)


@jax.jit
def kernel(x, Wg, bg, We, be):
    bg2 = bg.reshape(1, E)
    g, rank, ps_tok, xb, hist, loss = _tc1a(x, Wg, bg2)
    q2, ea, eb, nact = _tc1b(g, rank, hist)
    q = q2.reshape(NW * NCH, CH)
    xs, pss = _make_sc_scatter()(xb, ps_tok, q)
    comb = _tc2(xs, pss, ea, eb, nact, We, be)
    outs = _make_sc_gather()(comb, q)
    out = _tc3(outs)
    return out, loss.reshape(())


# speedup vs baseline: 1.3619x; 1.0540x over previous
import functools

import jax
import jax.numpy as jnp
from jax import lax
from jax.experimental import pallas as pl
from jax.experimental.pallas import tpu as pltpu
from jax.experimental.pallas import tpu_sc as plsc

N = 8192
D = 4096
E = 8
H = 512
BM = 256
M_BLOCKS = N // BM
NG = 64
BM2 = 128
GB = N // BM2 + 28
GBP = 128
T_MAX = GB * BM2

NW = 32
TPW = N // NW
CH = 8
NCH = TPW // CH


def _tc1a_kernel(x_ref, wg_ref, bg_ref,
                 g_ref, rank_ref, ps_ref, xb_ref, hist_ref, loss_ref,
                 carry_ref, psum_ref):
    m = pl.program_id(0)
    x_blk = x_ref[...]
    xb_blk = x_blk.astype(jnp.bfloat16)
    xb_ref[...] = pltpu.bitcast(
        xb_blk.reshape(2 * BM, D // 2), jnp.int32)

    logits = lax.dot_general(
        xb_blk, wg_ref[...].astype(jnp.bfloat16),
        dimension_numbers=(((1,), (0,)), ((), ())),
        preferred_element_type=jnp.float32) + bg_ref[...]
    mx = jnp.max(logits, axis=1, keepdims=True)
    ex = jnp.exp(logits - mx)
    p = ex / jnp.sum(ex, axis=1, keepdims=True)

    idx = lax.broadcasted_iota(jnp.int32, (BM, E), 1)
    v1 = jnp.max(p, axis=1, keepdims=True)
    i1 = jnp.min(jnp.where(p == v1, idx, E), axis=1, keepdims=True)
    p_m = jnp.where(idx == i1, -1.0, p)
    v2 = jnp.max(p_m, axis=1, keepdims=True)
    i2 = jnp.min(jnp.where(p_m == v2, idx, E), axis=1, keepdims=True)

    lo = jnp.minimum(i1, i2)
    hi = jnp.maximum(i1, i2)
    p_lo = jnp.where(i1 < i2, v1, v2)
    p_hi = jnp.where(i1 < i2, v2, v1)
    g = lo * E + hi

    @pl.when(m == 0)
    def _init():
        carry_ref[...] = jnp.zeros_like(carry_ref)
        psum_ref[...] = jnp.zeros_like(psum_ref)

    psum_ref[...] += jnp.sum(p, axis=0, keepdims=True)

    gcols = lax.broadcasted_iota(jnp.int32, (BM, NG), 1)
    ind = (g == gcols).astype(jnp.float32)
    tri = (lax.broadcasted_iota(jnp.int32, (BM, BM), 0) >
           lax.broadcasted_iota(jnp.int32, (BM, BM), 1)).astype(jnp.bfloat16)
    cnt_before = lax.dot_general(
        tri, ind.astype(jnp.bfloat16),
        dimension_numbers=(((1,), (0,)), ((), ())),
        preferred_element_type=jnp.float32)
    rank = jnp.sum((carry_ref[...] + cnt_before) * ind, axis=1, keepdims=True)
    carry_ref[...] += jnp.sum(ind, axis=0, keepdims=True)

    g_ref[...] = g
    rank_ref[...] = rank.astype(jnp.int32)
    ps_ref[...] = jnp.concatenate(
        [p_lo, p_hi, jnp.zeros((BM, 126), jnp.float32)], axis=1)

    @pl.when(m == M_BLOCKS - 1)
    def _fin():
        hist_ref[...] = carry_ref[...]
        mean_p = psum_ref[...] / N
        loss_ref[...] = jnp.sum((mean_p - 1.0 / E) ** 2, keepdims=True)


def _tc1a(x, Wg, bg2):
    return pl.pallas_call(
        _tc1a_kernel,
        grid=(M_BLOCKS,),
        in_specs=[
            pl.BlockSpec((BM, D), lambda m: (m, 0)),
            pl.BlockSpec((D, E), lambda m: (0, 0)),
            pl.BlockSpec((1, E), lambda m: (0, 0)),
        ],
        out_specs=[
            pl.BlockSpec((BM, 1), lambda m: (m, 0)),
            pl.BlockSpec((BM, 1), lambda m: (m, 0)),
            pl.BlockSpec((BM, 128), lambda m: (m, 0)),
            pl.BlockSpec((BM, D // 2), lambda m: (m, 0)),
            pl.BlockSpec((1, NG), lambda m: (0, 0)),
            pl.BlockSpec((1, 1), lambda m: (0, 0)),
        ],
        out_shape=[
            jax.ShapeDtypeStruct((N, 1), jnp.int32),
            jax.ShapeDtypeStruct((N, 1), jnp.int32),
            jax.ShapeDtypeStruct((N, 128), jnp.float32),
            jax.ShapeDtypeStruct((N, D // 2), jnp.int32),
            jax.ShapeDtypeStruct((1, NG), jnp.float32),
            jax.ShapeDtypeStruct((1, 1), jnp.float32),
        ],
        scratch_shapes=[
            pltpu.VMEM((1, NG), jnp.float32),
            pltpu.VMEM((1, E), jnp.float32),
        ],
        compiler_params=pltpu.CompilerParams(
            dimension_semantics=("arbitrary",)),
    )(x, Wg, bg2)


def _tc1b_kernel(g_ref, rank_ref, hist_ref, q_ref, ea_ref, eb_ref, nact_ref):
    m = pl.program_id(0)
    hist = hist_ref[...]
    nb = jnp.floor((hist + (BM2 - 1)) / BM2)
    ut = (lax.broadcasted_iota(jnp.int32, (NG, NG), 0) <
          lax.broadcasted_iota(jnp.int32, (NG, NG), 1)).astype(jnp.bfloat16)
    bstart = lax.dot_general(
        nb.astype(jnp.bfloat16), ut,
        dimension_numbers=(((1,), (0,)), ((), ())),
        preferred_element_type=jnp.float32)

    g = g_ref[...]
    gcols = lax.broadcasted_iota(jnp.int32, (BM, NG), 1)
    ind = (g == gcols).astype(jnp.float32)
    qpos = (jnp.sum(ind * bstart, axis=1, keepdims=True) * BM2
            + rank_ref[...].astype(jnp.float32))
    q_ref[...] = qpos.astype(jnp.int32)

    @pl.when(m == 0)
    def _desc():
        brow = lax.broadcasted_iota(jnp.int32, (GBP, NG), 0).astype(jnp.float32)
        act = jnp.logical_and(brow >= bstart, brow < bstart + nb)
        actf = act.astype(jnp.float32)
        glane = lax.broadcasted_iota(jnp.int32, (GBP, NG), 1).astype(jnp.float32)
        gb = jnp.sum(actf * glane, axis=1, keepdims=True)
        ea = jnp.floor(gb / E)
        eb = gb - ea * E
        na = jnp.sum(actf * jnp.clip(hist - (brow - bstart) * BM2,
                                     0.0, 1.0 * BM2),
                     axis=1, keepdims=True)
        ea_ref[...] = ea.astype(jnp.int32)
        eb_ref[...] = eb.astype(jnp.int32)
        nact_ref[...] = na.astype(jnp.int32)


def _tc1b(g, rank, hist):
    return pl.pallas_call(
        _tc1b_kernel,
        grid=(M_BLOCKS,),
        in_specs=[
            pl.BlockSpec((BM, 1), lambda m: (m, 0)),
            pl.BlockSpec((BM, 1), lambda m: (m, 0)),
            pl.BlockSpec((1, NG), lambda m: (0, 0)),
        ],
        out_specs=[
            pl.BlockSpec((BM, 1), lambda m: (m, 0)),
            pl.BlockSpec((GBP, 1), lambda m: (0, 0)),
            pl.BlockSpec((GBP, 1), lambda m: (0, 0)),
            pl.BlockSpec((GBP, 1), lambda m: (0, 0)),
        ],
        out_shape=[
            jax.ShapeDtypeStruct((N, 1), jnp.int32),
            jax.ShapeDtypeStruct((GBP, 1), jnp.int32),
            jax.ShapeDtypeStruct((GBP, 1), jnp.int32),
            jax.ShapeDtypeStruct((GBP, 1), jnp.int32),
        ],
        compiler_params=pltpu.CompilerParams(
            dimension_semantics=("arbitrary",)),
    )(g, rank, hist)


@functools.cache
def _make_sc_scatter():
    @functools.partial(
        pl.kernel,
        out_type=[
            jax.ShapeDtypeStruct((T_MAX, D // 2), jnp.int32),
            jax.ShapeDtypeStruct((T_MAX, 128), jnp.float32),
        ],
        mesh=plsc.VectorSubcoreMesh(core_axis_name="c", subcore_axis_name="s",
                                    num_cores=2, num_subcores=16),
        scratch_types=[
            pltpu.VMEM((NCH, CH), jnp.int32),
            pltpu.VMEM((2, CH, D // 2), jnp.int32),
            pltpu.VMEM((2, CH, 128), jnp.float32),
            pltpu.SemaphoreType.DMA,
            pltpu.SemaphoreType.DMA,
            pltpu.SemaphoreType.DMA,
            pltpu.SemaphoreType.DMA,
            pltpu.SemaphoreType.DMA,
        ],
    )
    def _sc_scatter(x_hbm, ps_hbm, q_hbm, xs_out, pss_out, qv, bufx, bufp,
                    semq, semld, semlp, semx, semp):
        wid = lax.axis_index("s") * 2 + lax.axis_index("c")
        base = wid * TPW
        cq = pltpu.async_copy(q_hbm.at[pl.ds(wid * NCH, NCH)], qv, semq)

        def load(i):
            b = i % 2
            off = base + i * CH
            pltpu.async_copy(x_hbm.at[pl.ds(off, CH)], bufx.at[b], semld)
            pltpu.async_copy(ps_hbm.at[pl.ds(off, CH)], bufp.at[b], semlp)

        load(0)
        cq.wait()
        for i in range(NCH):
            b = i % 2
            pltpu.make_async_copy(x_hbm.at[pl.ds(0, CH)], bufx.at[b],
                                  semld).wait()
            pltpu.make_async_copy(ps_hbm.at[pl.ds(0, CH)], bufp.at[b],
                                  semlp).wait()
            cx = pltpu.async_copy(bufx.at[b], xs_out.at[qv.at[i]], semx)
            cp = pltpu.async_copy(bufp.at[b], pss_out.at[qv.at[i]], semp)
            if i + 1 < NCH:
                load(i + 1)
            cx.wait()
            cp.wait()

    return _sc_scatter


def _tc2_kernel(xs_ref, ps_ref, ea_ref, eb_ref, nact_ref, we_hbm, be_ref,
                comb_ref, we_vmem, wtmp, sem):
    gi = pl.program_id(0)

    @pl.when(gi == 0)
    def _load():
        for e in range(E):
            cp = pltpu.make_async_copy(we_hbm.at[e], wtmp, sem)
            cp.start()
            cp.wait()
            we_vmem[e] = wtmp[...].astype(jnp.bfloat16)

    nact = nact_ref[gi, 0]

    @pl.when(nact > 0)
    def _compute():
        ea = ea_ref[gi, 0]
        eb = eb_ref[gi, 0]
        xsb = pltpu.bitcast(
            xs_ref[...], jnp.bfloat16).reshape(BM2, D)
        ps = ps_ref[...]
        pa = ps[:, 0:1]
        pb = ps[:, 1:2]

        mma = lax.dot_general(
            xsb, we_vmem[ea],
            dimension_numbers=(((1,), (0,)), ((), ())),
            preferred_element_type=jnp.float32)
        mmb = lax.dot_general(
            xsb, we_vmem[eb],
            dimension_numbers=(((1,), (0,)), ((), ())),
            preferred_element_type=jnp.float32)
        comb_ref[...] = (pa * (mma + be_ref[ea][None, :])
                         + pb * (mmb + be_ref[eb][None, :]))


def _tc2(xs, pss, ea, eb, nact, We, be):
    return pl.pallas_call(
        _tc2_kernel,
        grid=(GB,),
        in_specs=[
            pl.BlockSpec((BM2, D // 2), lambda m: (m, 0)),
            pl.BlockSpec((BM2, 128), lambda m: (m, 0)),
            pl.BlockSpec(memory_space=pltpu.MemorySpace.SMEM),
            pl.BlockSpec(memory_space=pltpu.MemorySpace.SMEM),
            pl.BlockSpec(memory_space=pltpu.MemorySpace.SMEM),
            pl.BlockSpec(memory_space=pl.ANY),
            pl.BlockSpec((E, H), lambda m: (0, 0)),
        ],
        out_specs=pl.BlockSpec((BM2, H), lambda m: (m, 0)),
        out_shape=jax.ShapeDtypeStruct((T_MAX, H), jnp.float32),
        scratch_shapes=[
            pltpu.VMEM((E, D, H), jnp.bfloat16),
            pltpu.VMEM((D, H), jnp.float32),
            pltpu.SemaphoreType.DMA,
        ],
        compiler_params=pltpu.CompilerParams(
            dimension_semantics=("arbitrary",)),
    )(xs, pss, ea, eb, nact, We, be)


@functools.cache
def _make_sc_gather():
    @functools.partial(
        pl.kernel,
        out_type=jax.ShapeDtypeStruct((N, H), jnp.float32),
        mesh=plsc.VectorSubcoreMesh(core_axis_name="c", subcore_axis_name="s",
                                    num_cores=2, num_subcores=16),
        scratch_types=[
            pltpu.VMEM((NCH, CH), jnp.int32),
            pltpu.VMEM((2, CH, H), jnp.float32),
            pltpu.SemaphoreType.DMA,
            pltpu.SemaphoreType.DMA,
            pltpu.SemaphoreType.DMA,
            pltpu.SemaphoreType.DMA,
        ],
    )
    def _sc_gather(comb_hbm, q_hbm, outs, qv, bufc, semq, semg, semw0, semw1):
        wid = lax.axis_index("s") * 2 + lax.axis_index("c")
        base = wid * TPW
        semw = (semw0, semw1)
        cq = pltpu.async_copy(q_hbm.at[pl.ds(wid * NCH, NCH)], qv, semq)
        cq.wait()
        pltpu.async_copy(comb_hbm.at[qv.at[0]], bufc.at[0], semg)
        for i in range(NCH):
            b = i % 2
            pltpu.make_async_copy(comb_hbm.at[pl.ds(0, CH)], bufc.at[b],
                                  semg).wait()
            pltpu.async_copy(bufc.at[b], outs.at[pl.ds(base + i * CH, CH)],
                             semw[b])
            if i + 1 < NCH:
                if i >= 1:
                    pltpu.make_async_copy(
                        comb_hbm.at[pl.ds(0, CH)], bufc.at[1 - b],
                        semw[1 - b]).wait()
                pltpu.async_copy(comb_hbm.at[qv.at[i + 1]], bufc.at[1 - b],
                                 semg)
        for b in (NCH % 2, (NCH + 1) % 2):
            pltpu.make_async_copy(comb_hbm.at[pl.ds(0, CH)], bufc.at[b],
                                  semw[b]).wait()

    return _sc_gather


def _tc3_kernel(os_ref, out_ref):
    out_ref[...] = jnp.tile(os_ref[...], (1, E))


def _tc3(outs):
    return pl.pallas_call(
        _tc3_kernel,
        grid=(M_BLOCKS,),
        in_specs=[pl.BlockSpec((BM, H), lambda m: (m, 0))],
        out_specs=pl.BlockSpec((BM, E * H), lambda m: (m, 0)),
        out_shape=jax.ShapeDtypeStruct((N, E * H), jnp.float32),
    )(outs)


@jax.jit
def kernel(x, Wg, bg, We, be):
    bg2 = bg.reshape(1, E)
    g, rank, ps_tok, xb, hist, loss = _tc1a(x, Wg, bg2)
    q2, ea, eb, nact = _tc1b(g, rank, hist)
    q = q2.reshape(NW * NCH, CH)
    xs, pss = _make_sc_scatter()(xb, ps_tok, q)
    comb = _tc2(xs, pss, ea, eb, nact, We, be)
    outs = _make_sc_gather()(comb, q)
    out = _tc3(outs)
    return out, loss.reshape(())
